# Initial kernel scaffold; baseline (speedup 1.0000x reference)
import jax
import jax.numpy as jnp
from jax.experimental import pallas as pl


def kernel(affinity_mat, aff_init, x_true, x_predict, z, z_pred, eta):
    raise NotImplementedError("write your pallas kernel here")


# speedup vs baseline: 9.4666x; 9.4666x over previous
import functools

import jax
import jax.numpy as jnp
from jax import lax
from jax.experimental import pallas as pl
from jax.experimental.pallas import tpu as pltpu
from jax.experimental.pallas import tpu_sc as plsc

_N = 4096
_K = 10
_INV_T = 10.0
_L = 16
_NC, _NS = 2, 16
_NW = _NC * _NS
_RPW = _N // _NW
_NCHUNK = _N // _L
_S = 8


def _merge_desc(av, ai, bv, bi):
    rv = lax.rev(bv, (0,))
    ri = lax.rev(bi, (0,))
    take = rv > av
    nv = jnp.where(take, rv, av)
    ni = jnp.where(take, ri, ai)
    return plsc.sort_key_val(nv, ni, descending=True)


def _sc_body(init_hbm, pred_hbm, out_hbm, init_v, pred_v, out_v):
    wid = lax.axis_index("s") * _NC + lax.axis_index("c")
    base = wid * _RPW
    lanes = lax.iota(jnp.int32, _L)

    def row_body(r, _):
        row = base + r
        pltpu.sync_copy(init_hbm.at[row], init_v)
        pltpu.sync_copy(pred_hbm.at[row], pred_v)

        def grp_body(g, carry):
            outs = []
            for s in range(_S):
                av, ai = carry[2 * s], carry[2 * s + 1]
                start = (g * _S + s) * _L
                v = init_v[pl.ds(start, _L)]
                cols = start + lanes
                v = jnp.where(cols == row, -1.0, v)
                sv, si = plsc.sort_key_val(v, cols, descending=False)
                take = sv > av
                nv = jnp.where(take, sv, av)
                ni = jnp.where(take, si, ai)
                av, ai = plsc.sort_key_val(nv, ni, descending=True)
                outs += [av, ai]
            return tuple(outs)

        carry = ()
        for _s in range(_S):
            carry += (jnp.full((_L,), -2.0, jnp.float32),
                      jnp.zeros((_L,), jnp.int32))
        carry = lax.fori_loop(0, _NCHUNK // _S, grp_body, carry)

        accs = [(carry[2 * s], carry[2 * s + 1]) for s in range(_S)]
        while len(accs) > 1:
            accs = [_merge_desc(*accs[i], *accs[i + 1])
                    for i in range(0, len(accs), 2)]
        _, fi = accs[0]
        out_v[...] = plsc.load_gather(pred_v, [fi])
        pltpu.sync_copy(out_v, out_hbm.at[row])
        return 0

    lax.fori_loop(0, _RPW, row_body, 0)


@functools.cache
def _sc_topk_gather():
    return functools.partial(
        pl.kernel,
        out_type=jax.ShapeDtypeStruct((_N, _L), jnp.float32),
        mesh=plsc.VectorSubcoreMesh(core_axis_name="c", subcore_axis_name="s",
                                    num_cores=_NC, num_subcores=_NS),
        scratch_types=[
            pltpu.VMEM((_N,), jnp.float32),
            pltpu.VMEM((_N,), jnp.float32),
            pltpu.VMEM((_L,), jnp.float32),
        ],
        compiler_params=pltpu.CompilerParams(needs_layout_passes=False),
    )(_sc_body)


_R = 128
_G = _N // _R


def _tc_body(aff, pp, xt, xp, z, zp, lm_ref, ls_ref, lr_ref, acc):
    g = pl.program_id(0)

    @pl.when(g == 0)
    def _():
        acc[0] = 0.0
        acc[1] = 0.0
        acc[2] = 0.0

    rows = g * _R + lax.broadcasted_iota(jnp.int32, (_R, _N), 0)
    cols = lax.broadcasted_iota(jnp.int32, (_R, _N), 1)
    sim = jnp.exp(aff[...] * _INV_T)
    sim = jnp.where(cols == rows, 0.0, sim)
    denom = jnp.sum(sim, axis=1, keepdims=True)

    pos = jnp.exp(pp[...] * _INV_T)
    terms = -jnp.log(pos / denom + 1e-8)
    kmask = lax.broadcasted_iota(jnp.int32, (_R, _L), 1) < _K
    sreg = jnp.sum(jnp.where(kmask, terms, 0.0))

    dx = xp[...] - xt[...]
    dz = zp[...] - z[...]
    acc[0] = acc[0] + sreg
    acc[1] = acc[1] + jnp.sum(dx * dx)
    acc[2] = acc[2] + jnp.sum(dz * dz)

    @pl.when(g == _G - 1)
    def _():
        lr_ref[0, 0] = acc[0] / _N
        lm_ref[0, 0] = acc[1] / (_N * 1024.0)
        ls_ref[0, 0] = acc[2] / (_N * 256.0)


_tc_loss = pl.pallas_call(
    _tc_body,
    grid=(_G,),
    in_specs=[
        pl.BlockSpec((_R, _N), lambda g: (g, 0)),
        pl.BlockSpec((_R, _L), lambda g: (g, 0)),
        pl.BlockSpec((_R, 1024), lambda g: (g, 0)),
        pl.BlockSpec((_R, 1024), lambda g: (g, 0)),
        pl.BlockSpec((_R, 256), lambda g: (g, 0)),
        pl.BlockSpec((_R, 256), lambda g: (g, 0)),
    ],
    out_specs=[
        pl.BlockSpec(memory_space=pltpu.SMEM),
        pl.BlockSpec(memory_space=pltpu.SMEM),
        pl.BlockSpec(memory_space=pltpu.SMEM),
    ],
    out_shape=[jax.ShapeDtypeStruct((1, 1), jnp.float32)] * 3,
    scratch_shapes=[pltpu.SMEM((3,), jnp.float32)],
)


def kernel(affinity_mat, aff_init, x_true, x_predict, z, z_pred, eta):
    pos_pred = _sc_topk_gather()(aff_init, affinity_mat)
    lm, ls, lr = _tc_loss(affinity_mat, pos_pred, x_true, x_predict, z, z_pred)
    lm = lm[0, 0]
    ls = ls[0, 0]
    lr = lr[0, 0]
    ene = jnp.exp(-eta)
    loss = jnp.sum(jnp.stack([lm, ls, lr]) * ene + eta)
    return (loss, lm, ls, lr, ene)


# speedup vs baseline: 11.1596x; 1.1788x over previous
import functools

import jax
import jax.numpy as jnp
from jax import lax
from jax.experimental import pallas as pl
from jax.experimental.pallas import tpu as pltpu
from jax.experimental.pallas import tpu_sc as plsc

_N = 4096
_K = 10
_INV_T = 10.0
_L = 16
_NC, _NS = 2, 16
_NW = _NC * _NS
_RPW = _N // _NW
_NCHUNK = _N // _L
_NGRP = _NCHUNK // _L


def _merge_desc(av, ai, bv, bi):
    rv = lax.rev(bv, (0,))
    ri = lax.rev(bi, (0,))
    take = rv > av
    nv = jnp.where(take, rv, av)
    ni = jnp.where(take, ri, ai)
    return plsc.sort_key_val(nv, ni, descending=True)


def _merge_unsorted(av, ai, v, i):
    sv, si = plsc.sort_key_val(v, i, descending=False)
    take = sv > av
    nv = jnp.where(take, sv, av)
    ni = jnp.where(take, si, ai)
    return plsc.sort_key_val(nv, ni, descending=True)


def _top16_of_vregs(pairs, n_acc):
    accs = []
    for a in range(n_acc):
        av = jnp.full((_L,), -2.0, jnp.float32)
        ai = jnp.zeros((_L,), jnp.int32)
        for p in range(a, len(pairs), n_acc):
            av, ai = _merge_unsorted(av, ai, *pairs[p])
        accs.append((av, ai))
    while len(accs) > 1:
        accs = [_merge_desc(*accs[i], *accs[i + 1])
                for i in range(0, len(accs), 2)]
    return accs[0]


def _sc_body(init_hbm, pred_flat_hbm, out_hbm,
             buf0, buf1, idx_blk, val_blk, sem0, sem1, gsem):
    wid = lax.axis_index("s") * _NC + lax.axis_index("c")
    base = wid * _RPW
    lanes = lax.iota(jnp.int32, _L)
    last = base + _RPW - 1

    pltpu.async_copy(init_hbm.at[base], buf0, sem0)
    pltpu.async_copy(init_hbm.at[base + 1], buf1, sem1)

    def do_row(row, buf, sem, nxt_sem):
        pltpu.make_async_copy(init_hbm.at[row], buf, sem).wait()
        plsc.store_scatter(buf, [jnp.full((_L,), row, jnp.int32)],
                           jnp.full((_L,), -1.0, jnp.float32),
                           mask=lanes == 0)

        grp_pairs = []
        for g in range(_NGRP):
            m = None
            for j in range(_L):
                e = plsc.load_gather(buf, [g * 256 + lanes * _L + j])
                m = e if m is None else jnp.maximum(m, e)
            grp_pairs.append((m, g * _L + lanes))
        _, cid = _top16_of_vregs(grp_pairs, 4)

        cbase = cid * _L
        cand_pairs = []
        for j in range(_L):
            cols = cbase + j
            cand_pairs.append((plsc.load_gather(buf, [cols]), cols))
        _, fcol = _top16_of_vregs(cand_pairs, 4)

        nxt = jnp.minimum(row + 2, last)
        pltpu.async_copy(init_hbm.at[nxt], buf, nxt_sem)

        r = row - base
        idx_blk[pl.ds(r * _L, _L)] = row * _N + fcol

    def pair_body(t, _):
        row0 = base + 2 * t
        do_row(row0, buf0, sem0, sem0)
        do_row(row0 + 1, buf1, sem1, sem1)
        return 0

    lax.fori_loop(0, _RPW // 2, pair_body, 0)
    pltpu.make_async_copy(init_hbm.at[last], buf0, sem0).wait()
    pltpu.make_async_copy(init_hbm.at[last], buf1, sem1).wait()

    descs = []
    for k in range(_RPW * _L // 128):
        descs.append(pltpu.async_copy(
            pred_flat_hbm.at[idx_blk.at[pl.ds(k * 128, 128)]],
            val_blk.at[pl.ds(k * 128, 128)], gsem))
    for d in descs:
        d.wait()
    pltpu.sync_copy(val_blk, out_hbm.at[pl.ds(base * _L, _RPW * _L)])


@functools.cache
def _sc_topk_gather():
    return functools.partial(
        pl.kernel,
        out_type=jax.ShapeDtypeStruct((_N * _L,), jnp.float32),
        mesh=plsc.VectorSubcoreMesh(core_axis_name="c", subcore_axis_name="s",
                                    num_cores=_NC, num_subcores=_NS),
        scratch_types=[
            pltpu.VMEM((_N,), jnp.float32),
            pltpu.VMEM((_N,), jnp.float32),
            pltpu.VMEM((_RPW * _L,), jnp.int32),
            pltpu.VMEM((_RPW * _L,), jnp.float32),
            pltpu.SemaphoreType.DMA,
            pltpu.SemaphoreType.DMA,
            pltpu.SemaphoreType.DMA,
        ],
        compiler_params=pltpu.CompilerParams(needs_layout_passes=False),
    )(_sc_body)


_R = 128
_G = _N // _R


def _tc_body(aff, pp, xt, xp, z, zp, lm_ref, ls_ref, lr_ref, acc):
    g = pl.program_id(0)

    @pl.when(g == 0)
    def _():
        acc[0] = 0.0
        acc[1] = 0.0
        acc[2] = 0.0

    rows = g * _R + lax.broadcasted_iota(jnp.int32, (_R, _N), 0)
    cols = lax.broadcasted_iota(jnp.int32, (_R, _N), 1)
    sim = jnp.exp(aff[...] * _INV_T)
    sim = jnp.where(cols == rows, 0.0, sim)
    denom = jnp.sum(sim, axis=1, keepdims=True)

    pos = jnp.exp(pp[...] * _INV_T)
    terms = -jnp.log(pos / denom + 1e-8)
    kmask = lax.broadcasted_iota(jnp.int32, (_R, _L), 1) < _K
    sreg = jnp.sum(jnp.where(kmask, terms, 0.0))

    dx = xp[...] - xt[...]
    dz = zp[...] - z[...]
    acc[0] = acc[0] + sreg
    acc[1] = acc[1] + jnp.sum(dx * dx)
    acc[2] = acc[2] + jnp.sum(dz * dz)

    @pl.when(g == _G - 1)
    def _():
        lr_ref[0, 0] = acc[0] / _N
        lm_ref[0, 0] = acc[1] / (_N * 1024.0)
        ls_ref[0, 0] = acc[2] / (_N * 256.0)


_tc_loss = pl.pallas_call(
    _tc_body,
    grid=(_G,),
    in_specs=[
        pl.BlockSpec((_R, _N), lambda g: (g, 0)),
        pl.BlockSpec((_R, _L), lambda g: (g, 0)),
        pl.BlockSpec((_R, 1024), lambda g: (g, 0)),
        pl.BlockSpec((_R, 1024), lambda g: (g, 0)),
        pl.BlockSpec((_R, 256), lambda g: (g, 0)),
        pl.BlockSpec((_R, 256), lambda g: (g, 0)),
    ],
    out_specs=[
        pl.BlockSpec(memory_space=pltpu.SMEM),
        pl.BlockSpec(memory_space=pltpu.SMEM),
        pl.BlockSpec(memory_space=pltpu.SMEM),
    ],
    out_shape=[jax.ShapeDtypeStruct((1, 1), jnp.float32)] * 3,
    scratch_shapes=[pltpu.SMEM((3,), jnp.float32)],
)


def kernel(affinity_mat, aff_init, x_true, x_predict, z, z_pred, eta):
    pos_pred = _sc_topk_gather()(aff_init, affinity_mat.reshape(-1))
    pos_pred = pos_pred.reshape(_N, _L)
    lm, ls, lr = _tc_loss(affinity_mat, pos_pred, x_true, x_predict, z, z_pred)
    lm = lm[0, 0]
    ls = ls[0, 0]
    lr = lr[0, 0]
    ene = jnp.exp(-eta)
    loss = jnp.sum(jnp.stack([lm, ls, lr]) * ene + eta)
    return (loss, lm, ls, lr, ene)
